# baseline (speedup 1.0000x reference)
import functools

import jax
import jax.numpy as jnp
from jax import lax
from jax.experimental import pallas as pl
from jax.experimental.pallas import tpu as pltpu
from jax.experimental.pallas import tpu_sc as plsc

TN = 1024
TH = 2048
LANES = 128


def _stage_logits(x_ref, wg_ref, bg_ref, l_ref, *, tn, e_num):
    xb = x_ref[0]
    wg = wg_ref[...]
    logits = lax.dot_general(wg, xb, (((1,), (1,)), ((), ())),
                             preferred_element_type=jnp.float32)
    l_ref[...] = logits + bg_ref[0][:, None]


def _sc_router(l_hbm, g_hbm, l_vmem, g_vmem, *, e_num, chunk):
    wid = lax.axis_index("s") * 2 + lax.axis_index("c")
    base = wid * chunk
    pltpu.sync_copy(l_hbm.at[:, pl.ds(base, chunk)], l_vmem)
    neg = jnp.full((16,), -3.0e38, jnp.float32)
    zero16 = jnp.zeros((16,), jnp.float32)
    for c in range(chunk // 16):
        sl = slice(c * 16, (c + 1) * 16)
        m1 = l_vmem[0, sl]
        i1 = jnp.zeros((16,), jnp.int32)
        m2 = neg
        i2 = jnp.zeros((16,), jnp.int32)
        for e in range(1, e_num):
            v = l_vmem[e, sl]
            es = jnp.full((16,), e, jnp.int32)
            new_top = v > m1
            beats2 = v > m2
            m2 = jnp.where(new_top, m1, jnp.where(beats2, v, m2))
            i2 = jnp.where(new_top, i1, jnp.where(beats2, es, i2))
            m1 = jnp.where(new_top, v, m1)
            i1 = jnp.where(new_top, es, i1)
        g1 = 1.0 / (1.0 + jnp.exp(m2 - m1))
        g2 = 1.0 - g1
        for e in range(e_num):
            es = jnp.full((16,), e, jnp.int32)
            g_vmem[e, sl] = jnp.where(i1 == es, g1,
                                      jnp.where(i2 == es, g2, zero16))
    pltpu.sync_copy(g_vmem, g_hbm.at[:, pl.ds(base, chunk)])


def _stage_s(x_ref, g_ref, s_ref, gsum_ref, *, tn, e_num):
    n = pl.program_id(1)
    xb = x_ref[0]
    gates = g_ref[...]
    sc = lax.dot_general(gates, xb, (((1,), (0,)), ((), ())),
                         preferred_element_type=jnp.float32)
    gs = jnp.sum(gates, axis=1)[None, None, :]

    @pl.when(n == 0)
    def _():
        s_ref[0] = sc
        gsum_ref[...] = gs

    @pl.when(n > 0)
    def _():
        s_ref[0] = s_ref[0] + sc
        gsum_ref[...] = gsum_ref[...] + gs


RG = 64


def _stage_b(wea_ref, web_ref, wec_ref, wed_ref, s_ref, be_ref, gsum_ref,
             out_ref, acc_ref, accb_ref, *, th, e_num, d, b_num):
    e = pl.program_id(1)
    be_blk = be_ref[pl.ds(e, 1), 0]
    gs = gsum_ref[:, 0, :]
    eids = lax.broadcasted_iota(jnp.int32, (b_num, e_num), 1)
    gse = jnp.sum(jnp.where(eids == e, gs, 0.0), axis=1)
    s0 = s_ref[pl.ds(e, 1), 0]
    s1 = s_ref[pl.ds(e + e_num, 1), 0]

    @pl.when(e == 0)
    def _():
        acc_ref[...] = jnp.zeros((b_num, th, LANES), jnp.float32)
        accb_ref[...] = jnp.zeros((b_num, th), jnp.float32)

    n_chunks = d // LANES
    quarter = th // 4
    wrefs = [wea_ref, web_ref, wec_ref, wed_ref]
    for r in range(th // RG):
        rows = slice(r * RG, (r + 1) * RG)
        q = (r * RG) // quarter
        w_ref = wrefs[q]
        wrows = slice(r * RG - q * quarter, (r + 1) * RG - q * quarter)
        acc0 = acc_ref[0, rows]
        acc1 = acc_ref[1, rows]
        for k in range(n_chunks):
            cols = slice(k * LANES, (k + 1) * LANES)
            wv = w_ref[0, wrows, cols]
            acc0 = acc0 + wv * s0[:, cols]
            acc1 = acc1 + wv * s1[:, cols]
        acc_ref[0, rows] = acc0
        acc_ref[1, rows] = acc1
    accb_ref[...] = accb_ref[...] + gse[:, None] * be_blk
    @pl.when(e == e_num - 1)
    def _():
        rows = [jnp.sum(acc_ref[b], axis=-1) + accb_ref[b]
                for b in range(b_num)]
        out_ref[...] = jnp.stack(rows, axis=0)


def kernel(x, Wg, bg, We, be):
    B, N, D = x.shape
    E, H, _ = We.shape
    tn = min(TN, N)
    th = min(TH, H)
    bg2 = bg.reshape(1, E)

    T = B * N
    nn = N // tn

    L = pl.pallas_call(
        functools.partial(_stage_logits, tn=tn, e_num=E),
        grid=(B, nn),
        in_specs=[
            pl.BlockSpec((1, tn, D), lambda b, n: (b, n, 0)),
            pl.BlockSpec((E, D), lambda b, n: (0, 0)),
            pl.BlockSpec((1, E), lambda b, n: (0, 0)),
        ],
        out_specs=pl.BlockSpec((E, tn), lambda b, n, _nn=nn: (0, b * _nn + n)),
        out_shape=jax.ShapeDtypeStruct((E, T), jnp.float32),
        compiler_params=pltpu.CompilerParams(
            dimension_semantics=("parallel", "arbitrary")),
    )(x, Wg, bg2)

    chunk = T // 32
    G = pl.kernel(
        functools.partial(_sc_router, e_num=E, chunk=chunk),
        out_type=jax.ShapeDtypeStruct((E, T), jnp.float32),
        mesh=plsc.VectorSubcoreMesh(core_axis_name="c", subcore_axis_name="s"),
        scratch_types=[
            pltpu.VMEM((E, chunk), jnp.float32),
            pltpu.VMEM((E, chunk), jnp.float32),
        ],
    )(L)

    S, Gsum = pl.pallas_call(
        functools.partial(_stage_s, tn=tn, e_num=E),
        grid=(B, nn),
        in_specs=[
            pl.BlockSpec((1, tn, D), lambda b, n: (b, n, 0)),
            pl.BlockSpec((E, tn), lambda b, n, _nn=nn: (0, b * _nn + n)),
        ],
        out_specs=[
            pl.BlockSpec((1, E, D), lambda b, n: (b, 0, 0)),
            pl.BlockSpec((1, 1, E), lambda b, n: (b, 0, 0)),
        ],
        out_shape=[
            jax.ShapeDtypeStruct((B, E, D), jnp.float32),
            jax.ShapeDtypeStruct((B, 1, E), jnp.float32),
        ],
        compiler_params=pltpu.CompilerParams(
            dimension_semantics=("parallel", "arbitrary")),
    )(x, G)

    S2 = S.reshape(B * E, 1, D)
    out = pl.pallas_call(
        functools.partial(_stage_b, th=th, e_num=E, d=D, b_num=B),
        grid=(H // th, E),
        in_specs=[
            pl.BlockSpec((1, th // 4, D), lambda h, e: (e, 4 * h, 0)),
            pl.BlockSpec((1, th // 4, D), lambda h, e: (e, 4 * h + 1, 0)),
            pl.BlockSpec((1, th // 4, D), lambda h, e: (e, 4 * h + 2, 0)),
            pl.BlockSpec((1, th // 4, D), lambda h, e: (e, 4 * h + 3, 0)),
            pl.BlockSpec((B * E, 1, D), lambda h, e: (0, 0, 0)),
            pl.BlockSpec((E, 1, th), lambda h, e: (0, 0, h)),
            pl.BlockSpec((B, 1, E), lambda h, e: (0, 0, 0)),
        ],
        out_specs=pl.BlockSpec((B, th), lambda h, e: (0, h)),
        out_shape=jax.ShapeDtypeStruct((B, H), jnp.float32),
        scratch_shapes=[
            pltpu.VMEM((B, th, LANES), jnp.float32),
            pltpu.VMEM((B, th), jnp.float32),
        ],
        compiler_params=pltpu.CompilerParams(
            dimension_semantics=("parallel", "arbitrary")),
    )(We, We, We, We, S2, be.reshape(E, 1, H), Gsum)
    return out
